# baseline (speedup 1.0000x reference)
import functools

import jax
import jax.numpy as jnp
from jax import lax
from jax.experimental import pallas as pl
from jax.experimental.pallas import tpu as pltpu
from jax.experimental.pallas import tpu_sc as plsc

SIZE = 1024
DIM = 256
N_TOKENS = 16384
LR_SCALE = 1.0

BLK = 512
P = 4
PIECE = N_TOKENS // P
NBP = PIECE // BLK

_NC = 2
_NS = 16
_NW = _NC * _NS
_ROWS_PER_W = PIECE // _NW
_CHUNK = 128
_NCHUNK = _ROWS_PER_W // _CHUNK


def _prep_body(ct_ref, ls_ref, cn_ref, expt_ref):
    expt_ref[...] = jnp.exp(ls_ref[...] * LR_SCALE)
    ct = ct_ref[...]
    cn_ref[...] = ct / jnp.maximum(
        jnp.sqrt(jnp.sum(ct * ct, axis=0, keepdims=True)), 1e-8)


def _prep_call(centers_t, log_std_weight):
    return pl.pallas_call(
        _prep_body,
        out_shape=[
            jax.ShapeDtypeStruct((DIM, SIZE), jnp.float32),
            jax.ShapeDtypeStruct((SIZE, DIM), jnp.float32),
        ],
    )(centers_t, log_std_weight)


def _tc_body(x_ref, cn_ref, idx_ref):
    x = x_ref[...]
    x = x / jnp.maximum(
        jnp.sqrt(jnp.sum(x * x, axis=1, keepdims=True)), 1e-8)
    s = jnp.dot(x, cn_ref[...], preferred_element_type=jnp.float32)
    m = jnp.max(s, axis=1, keepdims=True)
    cols = lax.broadcasted_iota(jnp.int32, s.shape, 1)
    idx = jnp.min(jnp.where(s == m, cols, SIZE), axis=1)
    idx_ref[...] = idx.reshape(1, 1, BLK)


def _tc_piece(x_piece, cn):
    return pl.pallas_call(
        _tc_body,
        grid=(NBP,),
        in_specs=[
            pl.BlockSpec((BLK, DIM), lambda i: (i, 0)),
            pl.BlockSpec((DIM, SIZE), lambda i: (0, 0)),
        ],
        out_specs=pl.BlockSpec((1, 1, BLK), lambda i: (i, 0, 0)),
        out_shape=jax.ShapeDtypeStruct((NBP, 1, BLK), jnp.int32),
    )(x_piece, cn)


@functools.cache
def _sc_gather(piece):
    @functools.partial(
        pl.kernel,
        out_type=[
            jax.ShapeDtypeStruct((PIECE, DIM), jnp.float32),
            jax.ShapeDtypeStruct((PIECE, DIM), jnp.float32),
        ],
        mesh=plsc.VectorSubcoreMesh(core_axis_name="c", subcore_axis_name="s",
                                    num_cores=_NC, num_subcores=_NS),
        scratch_types=[
            pltpu.VMEM((_CHUNK,), jnp.int32),
            pltpu.VMEM((_CHUNK, DIM), jnp.float32),
            pltpu.VMEM((_CHUNK, DIM), jnp.float32),
            pltpu.SemaphoreType.DMA,
            pltpu.SemaphoreType.DMA,
        ],
        name=f"sc_gather_p{piece}",
    )
    def _gather(mean_hbm, expt_hbm, idx_hbm, meanq_hbm, stdq_hbm,
                idx_v, mrows, srows, sem1, sem2):
        wid = lax.axis_index("s") * _NC + lax.axis_index("c")
        for ci in range(_NCHUNK):
            base = wid * _ROWS_PER_W + ci * _CHUNK
            pltpu.sync_copy(idx_hbm.at[pl.ds(base, _CHUNK)], idx_v)
            c1 = pltpu.async_copy(mean_hbm.at[idx_v], mrows, sem1)
            c2 = pltpu.async_copy(expt_hbm.at[idx_v], srows, sem2)
            c1.wait()
            c2.wait()
            pltpu.sync_copy(mrows, meanq_hbm.at[pl.ds(base, _CHUNK)])
            pltpu.sync_copy(srows, stdq_hbm.at[pl.ds(base, _CHUNK)])

    return _gather


def kernel(input_emb, mean_weight, log_std_weight):
    cn, expt = _prep_call(mean_weight.T, log_std_weight)
    idx_pieces, mq_pieces, sq_pieces = [], [], []
    for p in range(P):
        xp = lax.slice_in_dim(input_emb, p * PIECE, (p + 1) * PIECE, axis=0)
        idx_p = _tc_piece(xp, cn).reshape(PIECE)
        mq_p, sq_p = _sc_gather(p)(mean_weight, expt, idx_p)
        idx_pieces.append(idx_p)
        mq_pieces.append(mq_p)
        sq_pieces.append(sq_p)
    indices = jnp.concatenate(idx_pieces)
    mean_q = jnp.concatenate(mq_pieces)
    std_q = jnp.concatenate(sq_pieces)
    return indices, mean_q, std_q


# speedup vs baseline: 1.2975x; 1.2975x over previous
import functools

import jax
import jax.numpy as jnp
from jax import lax
from jax.experimental import pallas as pl
from jax.experimental.pallas import tpu as pltpu
from jax.experimental.pallas import tpu_sc as plsc

SIZE = 1024
DIM = 256
N_TOKENS = 16384
LR_SCALE = 1.0

BLK = 512
NB = N_TOKENS // BLK

_NC = 2
_NS = 16
_NW = _NC * _NS
_ROWS_PER_W = N_TOKENS // _NW
_CHUNK = 128
_NCHUNK = _ROWS_PER_W // _CHUNK


def _tc_body(x_ref, mw_ref, ls_ref, idx_ref, expt_ref, cn_ref):
    @pl.when(pl.program_id(0) == 0)
    def _():
        expt_ref[...] = jnp.exp(ls_ref[...] * LR_SCALE)
        c = mw_ref[...] * LR_SCALE
        cn_ref[...] = c / jnp.maximum(
            jnp.sqrt(jnp.sum(c * c, axis=1, keepdims=True)), 1e-8)

    x = x_ref[...]
    x = x / jnp.maximum(
        jnp.sqrt(jnp.sum(x * x, axis=1, keepdims=True)), 1e-8)
    s = lax.dot_general(x, cn_ref[...], (((1,), (1,)), ((), ())),
                        preferred_element_type=jnp.float32)
    m = jnp.max(s, axis=1, keepdims=True)
    cols = lax.broadcasted_iota(jnp.int32, s.shape, 1)
    idx = jnp.min(jnp.where(s == m, cols, SIZE), axis=1)
    idx_ref[...] = idx.reshape(1, 1, BLK)


def _tc_call(input_emb, mean_weight, log_std_weight):
    return pl.pallas_call(
        _tc_body,
        grid=(NB,),
        in_specs=[
            pl.BlockSpec((BLK, DIM), lambda i: (i, 0)),
            pl.BlockSpec((SIZE, DIM), lambda i: (0, 0)),
            pl.BlockSpec((SIZE, DIM), lambda i: (0, 0)),
        ],
        out_specs=[
            pl.BlockSpec((1, 1, BLK), lambda i: (i, 0, 0)),
            pl.BlockSpec((SIZE, DIM), lambda i: (0, 0)),
        ],
        out_shape=[
            jax.ShapeDtypeStruct((NB, 1, BLK), jnp.int32),
            jax.ShapeDtypeStruct((SIZE, DIM), jnp.float32),
        ],
        scratch_shapes=[pltpu.VMEM((SIZE, DIM), jnp.float32)],
    )(input_emb, mean_weight, log_std_weight)


@functools.cache
def _sc_gather():
    @functools.partial(
        pl.kernel,
        out_type=[
            jax.ShapeDtypeStruct((N_TOKENS, DIM), jnp.float32),
            jax.ShapeDtypeStruct((N_TOKENS, DIM), jnp.float32),
        ],
        mesh=plsc.VectorSubcoreMesh(core_axis_name="c", subcore_axis_name="s",
                                    num_cores=_NC, num_subcores=_NS),
        scratch_types=[
            pltpu.VMEM((_CHUNK,), jnp.int32),
            pltpu.VMEM((_CHUNK, DIM), jnp.float32),
            pltpu.VMEM((_CHUNK, DIM), jnp.float32),
            pltpu.SemaphoreType.DMA,
            pltpu.SemaphoreType.DMA,
        ],
        name="sc_gather",
    )
    def _gather(mean_hbm, expt_hbm, idx_hbm, meanq_hbm, stdq_hbm,
                idx_v, mrows, srows, sem1, sem2):
        wid = lax.axis_index("s") * _NC + lax.axis_index("c")
        for ci in range(_NCHUNK):
            base = wid * _ROWS_PER_W + ci * _CHUNK
            pltpu.sync_copy(idx_hbm.at[pl.ds(base, _CHUNK)], idx_v)
            c1 = pltpu.async_copy(mean_hbm.at[idx_v], mrows, sem1)
            c2 = pltpu.async_copy(expt_hbm.at[idx_v], srows, sem2)
            c1.wait()
            c2.wait()
            pltpu.sync_copy(mrows, meanq_hbm.at[pl.ds(base, _CHUNK)])
            pltpu.sync_copy(srows, stdq_hbm.at[pl.ds(base, _CHUNK)])

    return _gather


def kernel(input_emb, mean_weight, log_std_weight):
    idx3, expt = _tc_call(input_emb, mean_weight, log_std_weight)
    indices = idx3.reshape(N_TOKENS)
    mean_q, std_q = _sc_gather()(mean_weight, expt, indices)
    return indices, mean_q, std_q
